# Initial kernel scaffold; baseline (speedup 1.0000x reference)
import jax
import jax.numpy as jnp
from jax.experimental import pallas as pl


def kernel(fts, adj, W1, b1, W2, b2):
    raise NotImplementedError("write your pallas kernel here")


# speedup vs baseline: 1.3541x; 1.3541x over previous
import functools

import jax
import jax.numpy as jnp
from jax.experimental import pallas as pl
from jax.experimental.pallas import tpu as pltpu


def _sage_layer_body(adj_ref, xa_ref, xs_ref, ws_ref, wn_ref, b_ref, out_ref,
                     *, feat, apply_relu):
    a = adj_ref[...].astype(jnp.bfloat16)
    prod = jnp.dot(a, xa_ref[...], preferred_element_type=jnp.float32)
    s = prod[:, :feat]
    deg = jnp.clip(prod[:, feat:feat + 1], 1e-6, None)
    neigh = s / deg
    h = (jnp.dot(xs_ref[...], ws_ref[...], preferred_element_type=jnp.float32,
                 precision=jax.lax.Precision.HIGHEST)
         + jnp.dot(neigh, wn_ref[...], preferred_element_type=jnp.float32,
                   precision=jax.lax.Precision.HIGHEST)
         + b_ref[...])
    if apply_relu:
        h = jnp.maximum(h, 0.0)
    out_ref[...] = h


def _pick_bm(n):
    for c in (400, 256, 250, 200, 128, 100, 80, 64, 50, 40, 32, 25, 20, 16,
              10, 8, 5, 4, 2, 1):
        if n % c == 0:
            return c
    return n


def _sage_layer(adj, x, w, b, apply_relu):
    n = adj.shape[0]
    feat = x.shape[1]
    bm = _pick_bm(n)
    xa = jnp.concatenate(
        [x.astype(jnp.bfloat16), jnp.ones((n, 1), jnp.bfloat16)], axis=1)
    ws = w[:feat]
    wn = w[feat:]
    b2 = b.reshape(1, feat)
    body = functools.partial(_sage_layer_body, feat=feat, apply_relu=apply_relu)
    return pl.pallas_call(
        body,
        grid=(n // bm,),
        in_specs=[
            pl.BlockSpec((bm, n), lambda i: (i, 0)),
            pl.BlockSpec((n, feat + 1), lambda i: (0, 0)),
            pl.BlockSpec((bm, feat), lambda i: (i, 0)),
            pl.BlockSpec((feat, feat), lambda i: (0, 0)),
            pl.BlockSpec((feat, feat), lambda i: (0, 0)),
            pl.BlockSpec((1, feat), lambda i: (0, 0)),
        ],
        out_specs=pl.BlockSpec((bm, feat), lambda i: (i, 0)),
        out_shape=jax.ShapeDtypeStruct((n, feat), jnp.float32),
        compiler_params=pltpu.CompilerParams(
            dimension_semantics=("arbitrary",),
        ),
    )(adj, xa, x, ws, wn, b2)


def kernel(fts, adj, W1, b1, W2, b2):
    h = _sage_layer(adj, fts, W1, b1, apply_relu=True)
    return _sage_layer(adj, h, W2, b2, apply_relu=False)
